# baseline (speedup 1.0000x reference)
import jax
import jax.numpy as jnp
from jax.experimental import pallas as pl

_BLOCK = 1000


def _mlp_body(x_ref, w1, b1, w2, b2, w3, b3, w4, b4, w5, b5, w6, b6, o_ref):
    f32 = jnp.float32
    y = x_ref[...]
    y = jnp.maximum(jnp.dot(y, w1[...], preferred_element_type=f32) + b1[...], 0.0)
    y = jnp.maximum(jnp.dot(y, w2[...], preferred_element_type=f32) + b2[...], 0.0)
    y = jnp.maximum(jnp.dot(y, w3[...], preferred_element_type=f32) + b3[...], 0.0)
    y = jnp.maximum(jnp.dot(y, w4[...], preferred_element_type=f32) + b4[...], 0.0)
    y = jnp.maximum(jnp.dot(y, w5[...], preferred_element_type=f32) + b5[...], 0.0)
    o_ref[...] = jnp.dot(y, w6[...], preferred_element_type=f32) + b6[...]


def kernel(x_coord, edge_index, W1, b1, W2, b2, W3, b3, W4, b4, W5, b5, W6, b6):
    del edge_index
    n, d_in = x_coord.shape
    d_out = W6.shape[1]

    ws = [W1, W2, W3, W4, W5, W6]
    bs = [b.reshape(1, -1) for b in (b1, b2, b3, b4, b5, b6)]

    operands = []
    in_specs = [pl.BlockSpec((_BLOCK, d_in), lambda i: (i, 0))]
    for w, b in zip(ws, bs):
        operands.extend([w, b])
        in_specs.append(pl.BlockSpec(w.shape, lambda i: (0, 0)))
        in_specs.append(pl.BlockSpec(b.shape, lambda i: (0, 0)))

    return pl.pallas_call(
        _mlp_body,
        grid=(n // _BLOCK,),
        in_specs=in_specs,
        out_specs=pl.BlockSpec((_BLOCK, d_out), lambda i: (i, 0)),
        out_shape=jax.ShapeDtypeStruct((n, d_out), jnp.float32),
    )(x_coord, *operands)


# speedup vs baseline: 1.4175x; 1.4175x over previous
import jax
import jax.numpy as jnp
from jax.experimental import pallas as pl

_BLOCK = 2000


def _mlp_body(x_ref, w1, b1, w2, b2, w3, b3, w4, b4, w5, b5, w6, b6, o_ref):
    f32 = jnp.float32
    y = x_ref[...]
    y = jnp.maximum(jnp.dot(y, w1[...], preferred_element_type=f32) + b1[...], 0.0)
    y = jnp.maximum(jnp.dot(y, w2[...], preferred_element_type=f32) + b2[...], 0.0)
    y = jnp.maximum(jnp.dot(y, w3[...], preferred_element_type=f32) + b3[...], 0.0)
    y = jnp.maximum(jnp.dot(y, w4[...], preferred_element_type=f32) + b4[...], 0.0)
    y = jnp.maximum(jnp.dot(y, w5[...], preferred_element_type=f32) + b5[...], 0.0)
    o_ref[...] = jnp.dot(y, w6[...], preferred_element_type=f32) + b6[...]


def kernel(x_coord, edge_index, W1, b1, W2, b2, W3, b3, W4, b4, W5, b5, W6, b6):
    del edge_index
    n, d_in = x_coord.shape
    d_out = W6.shape[1]

    ws = [W1, W2, W3, W4, W5, W6]
    bs = [b.reshape(1, -1) for b in (b1, b2, b3, b4, b5, b6)]

    operands = []
    in_specs = [pl.BlockSpec((_BLOCK, d_in), lambda i: (i, 0))]
    for w, b in zip(ws, bs):
        operands.extend([w, b])
        in_specs.append(pl.BlockSpec(w.shape, lambda i: (0, 0)))
        in_specs.append(pl.BlockSpec(b.shape, lambda i: (0, 0)))

    return pl.pallas_call(
        _mlp_body,
        grid=(n // _BLOCK,),
        in_specs=in_specs,
        out_specs=pl.BlockSpec((_BLOCK, d_out), lambda i: (i, 0)),
        out_shape=jax.ShapeDtypeStruct((n, d_out), jnp.float32),
    )(x_coord, *operands)
